# baseline (speedup 1.0000x reference)
import dataclasses
import functools

import jax
import jax.numpy as jnp
from jax import lax
from jax.experimental import pallas as pl
from jax.experimental.pallas import tpu as pltpu
from jax.experimental.pallas import tpu_sc as plsc

K = 100000
D = 128

S = 60000
KSC = K - S

NC = 2
NS = 16
NW = NC * NS
RPW = KSC // NW
NBUF = 5
CH = 125
NROUND = RPW // (NBUF * CH)
assert RPW == NBUF * CH * NROUND
U = 5
NV = D // 16

BT = 2000
NT = S // BT
assert S == BT * NT


def _sc_l1_argmin(keys_flat, query_flat):
    mesh = plsc.VectorSubcoreMesh(core_axis_name="c", subcore_axis_name="s")
    cp = pltpu.CompilerParams()
    if "needs_layout_passes" in pltpu.CompilerParams.__dataclass_fields__:
        cp = dataclasses.replace(cp, needs_layout_passes=False)

    @functools.partial(
        pl.kernel,
        mesh=mesh,
        compiler_params=cp,
        out_type=[
            jax.ShapeDtypeStruct((NW, 16), jnp.float32),
            jax.ShapeDtypeStruct((NW, 16), jnp.int32),
        ],
        scratch_types=[pltpu.VMEM((CH * D,), jnp.float32)] * NBUF + [
            pltpu.VMEM((D,), jnp.float32),
            pltpu.VMEM((16,), jnp.float32),
            pltpu.VMEM((16,), jnp.int32),
        ] + [pltpu.SemaphoreType.DMA] * NBUF,
    )
    def k(keys_hbm, q_hbm, out_v_hbm, out_i_hbm, *rest):
        bufs = rest[:NBUF]
        q_v, res_v, resi_v = rest[NBUF:NBUF + 3]
        sems = rest[NBUF + 3:]
        wid = lax.axis_index("s") * NC + lax.axis_index("c")
        base = S + wid * RPW
        pltpu.sync_copy(q_hbm, q_v)
        qs = [q_v[pl.ds(16 * j, 16)] for j in range(NV)]

        def start(g, b):
            pltpu.async_copy(
                keys_hbm.at[pl.ds((base + g * CH) * D, CH * D)],
                bufs[b], sems[b])

        def wait(b):
            pltpu.make_async_copy(
                keys_hbm.at[pl.ds(0, CH * D)], bufs[b], sems[b]).wait()

        for b in range(NBUF):
            start(b, b)

        def chunk_body(i, carry, b):
            bv, bi = carry
            g = i * NBUF + b
            wait(b)
            gbase = base + g * CH

            def body(r5, carry):
                bv, bi = carry
                for u in range(U):
                    r = r5 * U + u
                    d = [jnp.abs(bufs[b][pl.ds(r * D + 16 * j, 16)] - qs[j])
                         for j in range(NV)]
                    s1 = [d[0] + d[1], d[2] + d[3], d[4] + d[5], d[6] + d[7]]
                    acc = (s1[0] + s1[1]) + (s1[2] + s1[3])
                    s = jnp.sum(acc)
                    pred = s < bv
                    bv = jnp.where(pred, s, bv)
                    bi = jnp.where(pred, gbase + r, bi)
                return bv, bi

            bv, bi = lax.fori_loop(0, CH // U, body, (bv, bi))

            @pl.when(i < NROUND - 1)
            def _():
                start(g + NBUF, b)

            return bv, bi

        def round_body(i, carry):
            for b in range(NBUF):
                carry = chunk_body(i, carry, b)
            return carry

        bv, bi = lax.fori_loop(
            0, NROUND, round_body, (jnp.float32(jnp.inf), jnp.int32(0)))

        res_v[...] = jnp.full((16,), bv, jnp.float32)
        resi_v[...] = jnp.full((16,), bi, jnp.int32)
        pltpu.sync_copy(res_v, out_v_hbm.at[wid])
        pltpu.sync_copy(resi_v, out_i_hbm.at[wid])

    return k(keys_flat, query_flat)


def _tc_l1_argmin(keys, query):
    def body(kany, kref, qref, ov, oi, bv_s, bs_s, blk, sem):
        step = pl.program_id(0)

        @pl.when(step == 0)
        def _():
            bv_s[0] = jnp.float32(jnp.inf)
            bs_s[0] = jnp.int32(0)

        d = jnp.sum(jnp.abs(kref[...] - qref[...]), axis=1, keepdims=True)
        m = jnp.min(d)
        pred = m < bv_s[0]
        bv_s[0] = jnp.where(pred, m, bv_s[0])
        bs_s[0] = jnp.where(pred, step, bs_s[0])

        @pl.when(step == NT - 1)
        def _():
            bs = bs_s[0]
            pltpu.async_copy(kany.at[pl.ds(bs * BT, BT), :], blk, sem).wait()
            d2 = jnp.sum(jnp.abs(blk[...] - qref[...]), axis=1, keepdims=True)
            iota = lax.broadcasted_iota(jnp.int32, (BT, 1), 0) + bs * BT
            eq = d2 == bv_s[0]
            ov[0] = bv_s[0]
            oi[0] = jnp.min(jnp.where(eq, iota, jnp.int32(2 ** 30)))

    return pl.pallas_call(
        body,
        grid=(NT,),
        in_specs=[pl.BlockSpec(memory_space=pl.ANY),
                  pl.BlockSpec((BT, D), lambda i: (i, 0)),
                  pl.BlockSpec((1, D), lambda i: (0, 0))],
        out_specs=[pl.BlockSpec(memory_space=pltpu.SMEM),
                   pl.BlockSpec(memory_space=pltpu.SMEM)],
        out_shape=[jax.ShapeDtypeStruct((1,), jnp.float32),
                   jax.ShapeDtypeStruct((1,), jnp.int32)],
        scratch_shapes=[pltpu.SMEM((1,), jnp.float32),
                        pltpu.SMEM((1,), jnp.int32),
                        pltpu.VMEM((BT, D), jnp.float32),
                        pltpu.SemaphoreType.DMA],
    )(keys, keys, query)


def kernel(keys, query):
    sc_v, sc_i = _sc_l1_argmin(keys.reshape((K * D,)), query.reshape((D,)))
    tc_v, tc_i = _tc_l1_argmin(keys, query)
    sv = sc_v[:, 0]
    si = sc_i[:, 0]
    m = jnp.min(sv)
    mi = jnp.min(jnp.where(sv == m, si, jnp.int32(2 ** 30)))
    pred = (tc_v[0] < m) | ((tc_v[0] == m) & (tc_i[0] < mi))
    return jnp.where(pred, tc_i[0], mi)


# speedup vs baseline: 1.0865x; 1.0865x over previous
import dataclasses
import functools

import jax
import jax.numpy as jnp
from jax import lax
from jax.experimental import pallas as pl
from jax.experimental.pallas import tpu as pltpu
from jax.experimental.pallas import tpu_sc as plsc

K = 100000
D = 128

S = 64000
KSC = K - S

NC = 2
NS = 16
NW = NC * NS
RPW = KSC // NW
NBUF = 5
CH = 75
NROUND = RPW // (NBUF * CH)
assert RPW == NBUF * CH * NROUND
U = 5
NV = D // 16

BT = 2560
NT = S // BT
assert S == BT * NT
TPB = BT // 128


def _sc_l1_argmin(keys_flat, query_flat):
    mesh = plsc.VectorSubcoreMesh(core_axis_name="c", subcore_axis_name="s")
    cp = pltpu.CompilerParams()
    if "needs_layout_passes" in pltpu.CompilerParams.__dataclass_fields__:
        cp = dataclasses.replace(cp, needs_layout_passes=False)

    @functools.partial(
        pl.kernel,
        mesh=mesh,
        compiler_params=cp,
        out_type=[
            jax.ShapeDtypeStruct((NW, 16), jnp.float32),
            jax.ShapeDtypeStruct((NW, 16), jnp.int32),
        ],
        scratch_types=[pltpu.VMEM((CH * D,), jnp.float32)] * NBUF + [
            pltpu.VMEM((D,), jnp.float32),
            pltpu.VMEM((16,), jnp.float32),
            pltpu.VMEM((16,), jnp.int32),
        ] + [pltpu.SemaphoreType.DMA] * NBUF,
    )
    def k(keys_hbm, q_hbm, out_v_hbm, out_i_hbm, *rest):
        bufs = rest[:NBUF]
        q_v, res_v, resi_v = rest[NBUF:NBUF + 3]
        sems = rest[NBUF + 3:]
        wid = lax.axis_index("s") * NC + lax.axis_index("c")
        base = S + wid * RPW
        pltpu.sync_copy(q_hbm, q_v)
        qs = [q_v[pl.ds(16 * j, 16)] for j in range(NV)]

        def start(g, b):
            pltpu.async_copy(
                keys_hbm.at[pl.ds((base + g * CH) * D, CH * D)],
                bufs[b], sems[b])

        def wait(b):
            pltpu.make_async_copy(
                keys_hbm.at[pl.ds(0, CH * D)], bufs[b], sems[b]).wait()

        for b in range(NBUF):
            start(b, b)

        def chunk_body(i, carry, b):
            bv, bi = carry
            g = i * NBUF + b
            wait(b)
            gbase = base + g * CH

            def body(r5, carry):
                bv, bi = carry
                for u in range(U):
                    r = r5 * U + u
                    d = [jnp.abs(bufs[b][pl.ds(r * D + 16 * j, 16)] - qs[j])
                         for j in range(NV)]
                    s1 = [d[0] + d[1], d[2] + d[3], d[4] + d[5], d[6] + d[7]]
                    acc = (s1[0] + s1[1]) + (s1[2] + s1[3])
                    s = jnp.sum(acc)
                    pred = s < bv
                    bv = jnp.where(pred, s, bv)
                    bi = jnp.where(pred, gbase + r, bi)
                return bv, bi

            bv, bi = lax.fori_loop(0, CH // U, body, (bv, bi))

            @pl.when(i < NROUND - 1)
            def _():
                start(g + NBUF, b)

            return bv, bi

        def round_body(i, carry):
            for b in range(NBUF):
                carry = chunk_body(i, carry, b)
            return carry

        bv, bi = lax.fori_loop(
            0, NROUND, round_body, (jnp.float32(jnp.inf), jnp.int32(0)))

        res_v[...] = jnp.full((16,), bv, jnp.float32)
        resi_v[...] = jnp.full((16,), bi, jnp.int32)
        pltpu.sync_copy(res_v, out_v_hbm.at[wid])
        pltpu.sync_copy(resi_v, out_i_hbm.at[wid])

    return k(keys_flat, query_flat)


def _tc_l1_argmin(keys, query):
    def body(kref, qref, ov, oi, best_v, best_i):
        step = pl.program_id(0)

        @pl.when(step == 0)
        def _():
            best_v[...] = jnp.full((1, D), jnp.inf, jnp.float32)
            best_i[...] = jnp.zeros((1, D), jnp.int32)

        q = qref[...]
        bv = best_v[...]
        bi = best_i[...]
        lane = lax.broadcasted_iota(jnp.int32, (1, D), 1)
        for t in range(TPB):
            a = jnp.abs(kref[pl.ds(t * 128, 128), :] - q)
            s = jnp.sum(a.T, axis=0, keepdims=True)
            idx = lane + (step * BT + t * 128)
            pred = s < bv
            bv = jnp.where(pred, s, bv)
            bi = jnp.where(pred, idx, bi)
        best_v[...] = bv
        best_i[...] = bi

        @pl.when(step == NT - 1)
        def _():
            m = jnp.min(bv)
            ci = jnp.min(jnp.where(bv == m, bi, jnp.int32(2 ** 30)))
            ov[0] = m
            oi[0] = ci

    return pl.pallas_call(
        body,
        grid=(NT,),
        in_specs=[pl.BlockSpec((BT, D), lambda i: (i, 0)),
                  pl.BlockSpec((1, D), lambda i: (0, 0))],
        out_specs=[pl.BlockSpec(memory_space=pltpu.SMEM),
                   pl.BlockSpec(memory_space=pltpu.SMEM)],
        out_shape=[jax.ShapeDtypeStruct((1,), jnp.float32),
                   jax.ShapeDtypeStruct((1,), jnp.int32)],
        scratch_shapes=[pltpu.VMEM((1, D), jnp.float32),
                        pltpu.VMEM((1, D), jnp.int32)],
    )(keys, query)


def kernel(keys, query):
    sc_v, sc_i = _sc_l1_argmin(keys.reshape((K * D,)), query.reshape((D,)))
    tc_v, tc_i = _tc_l1_argmin(keys, query)
    sv = sc_v[:, 0]
    si = sc_i[:, 0]
    m = jnp.min(sv)
    mi = jnp.min(jnp.where(sv == m, si, jnp.int32(2 ** 30)))
    pred = (tc_v[0] < m) | ((tc_v[0] == m) & (tc_i[0] < mi))
    return jnp.where(pred, tc_i[0], mi)


# speedup vs baseline: 1.0893x; 1.0026x over previous
import dataclasses
import functools

import jax
import jax.numpy as jnp
from jax import lax
from jax.experimental import pallas as pl
from jax.experimental.pallas import tpu as pltpu
from jax.experimental.pallas import tpu_sc as plsc

K = 100000
D = 128

S = 64000
KSC = K - S

NC = 2
NS = 16
NW = NC * NS
RPW = KSC // NW
NBUF = 3
CH = 75
NROUND = RPW // (NBUF * CH)
assert RPW == NBUF * CH * NROUND
U = 5
NV = D // 16

BT = 2560
NT = S // BT
assert S == BT * NT
TPB = BT // 128


def _sc_l1_argmin(keys_flat, query_flat):
    mesh = plsc.VectorSubcoreMesh(core_axis_name="c", subcore_axis_name="s")
    cp = pltpu.CompilerParams()
    if "needs_layout_passes" in pltpu.CompilerParams.__dataclass_fields__:
        cp = dataclasses.replace(cp, needs_layout_passes=False)

    @functools.partial(
        pl.kernel,
        mesh=mesh,
        compiler_params=cp,
        out_type=[
            jax.ShapeDtypeStruct((NW, 16), jnp.float32),
            jax.ShapeDtypeStruct((NW, 16), jnp.int32),
        ],
        scratch_types=[pltpu.VMEM((CH * D,), jnp.float32)] * NBUF + [
            pltpu.VMEM((D,), jnp.float32),
            pltpu.VMEM((16,), jnp.float32),
            pltpu.VMEM((16,), jnp.int32),
        ] + [pltpu.SemaphoreType.DMA] * NBUF,
    )
    def k(keys_hbm, q_hbm, out_v_hbm, out_i_hbm, *rest):
        bufs = rest[:NBUF]
        q_v, res_v, resi_v = rest[NBUF:NBUF + 3]
        sems = rest[NBUF + 3:]
        wid = lax.axis_index("s") * NC + lax.axis_index("c")
        base = S + wid * RPW
        pltpu.sync_copy(q_hbm, q_v)
        qs = [q_v[pl.ds(16 * j, 16)] for j in range(NV)]

        def start(g, b):
            pltpu.async_copy(
                keys_hbm.at[pl.ds((base + g * CH) * D, CH * D)],
                bufs[b], sems[b])

        def wait(b):
            pltpu.make_async_copy(
                keys_hbm.at[pl.ds(0, CH * D)], bufs[b], sems[b]).wait()

        for b in range(NBUF):
            start(b, b)

        def chunk_body(i, carry, b):
            bv, bi = carry
            g = i * NBUF + b
            wait(b)
            gbase = base + g * CH

            def body(r5, carry):
                bv, bi = carry
                for u in range(U):
                    r = r5 * U + u
                    d = [jnp.abs(bufs[b][pl.ds(r * D + 16 * j, 16)] - qs[j])
                         for j in range(NV)]
                    s1 = [d[0] + d[1], d[2] + d[3], d[4] + d[5], d[6] + d[7]]
                    acc = (s1[0] + s1[1]) + (s1[2] + s1[3])
                    s = jnp.sum(acc)
                    pred = s < bv
                    bv = jnp.where(pred, s, bv)
                    bi = jnp.where(pred, gbase + r, bi)
                return bv, bi

            bv, bi = lax.fori_loop(0, CH // U, body, (bv, bi))

            @pl.when(i < NROUND - 1)
            def _():
                start(g + NBUF, b)

            return bv, bi

        def round_body(i, carry):
            for b in range(NBUF):
                carry = chunk_body(i, carry, b)
            return carry

        bv, bi = lax.fori_loop(
            0, NROUND, round_body, (jnp.float32(jnp.inf), jnp.int32(0)))

        res_v[...] = jnp.full((16,), bv, jnp.float32)
        resi_v[...] = jnp.full((16,), bi, jnp.int32)
        pltpu.sync_copy(res_v, out_v_hbm.at[wid])
        pltpu.sync_copy(resi_v, out_i_hbm.at[wid])

    return k(keys_flat, query_flat)


NACC = 4
HB = BT // 2


def _tc_l1_argmin(keys, query):
    def body(kref0, kref1, qref, ov, oi, best_v, best_i):
        step = pl.program_id(0)

        @pl.when(step == 0)
        def _():
            for a in range(NACC):
                best_v[a, :] = jnp.full((128,), jnp.inf, jnp.float32)
                best_i[a, :] = jnp.zeros((128,), jnp.int32)

        q = qref[...]
        lane = lax.broadcasted_iota(jnp.int32, (1, D), 1)
        bv = [best_v[a, :].reshape(1, D) for a in range(NACC)]
        bi = [best_i[a, :].reshape(1, D) for a in range(NACC)]
        for t in range(TPB):
            kref = kref0 if t < TPB // 2 else kref1
            to = t if t < TPB // 2 else t - TPB // 2
            a = jnp.abs(kref[pl.ds(to * 128, 128), :] - q)
            s = jnp.sum(a.T, axis=0, keepdims=True)
            idx = lane + (step * BT + t * 128)
            c = t % NACC
            pred = s < bv[c]
            bv[c] = jnp.where(pred, s, bv[c])
            bi[c] = jnp.where(pred, idx, bi[c])
        for a in range(NACC):
            best_v[a, :] = bv[a].reshape(D)
            best_i[a, :] = bi[a].reshape(D)

        @pl.when(step == NT - 1)
        def _():
            mv, mi = bv[0], bi[0]
            for a in range(1, NACC):
                p = (bv[a] < mv) | ((bv[a] == mv) & (bi[a] < mi))
                mv = jnp.where(p, bv[a], mv)
                mi = jnp.where(p, bi[a], mi)
            m = jnp.min(mv)
            ci = jnp.min(jnp.where(mv == m, mi, jnp.int32(2 ** 30)))
            ov[0] = m
            oi[0] = ci

    return pl.pallas_call(
        body,
        grid=(NT,),
        in_specs=[pl.BlockSpec((HB, D), lambda i: (2 * i, 0)),
                  pl.BlockSpec((HB, D), lambda i: (2 * i + 1, 0)),
                  pl.BlockSpec((1, D), lambda i: (0, 0))],
        out_specs=[pl.BlockSpec(memory_space=pltpu.SMEM),
                   pl.BlockSpec(memory_space=pltpu.SMEM)],
        out_shape=[jax.ShapeDtypeStruct((1,), jnp.float32),
                   jax.ShapeDtypeStruct((1,), jnp.int32)],
        scratch_shapes=[pltpu.VMEM((NACC, D), jnp.float32),
                        pltpu.VMEM((NACC, D), jnp.int32)],
    )(keys, keys, query)


def kernel(keys, query):
    sc_v, sc_i = _sc_l1_argmin(keys.reshape((K * D,)), query.reshape((D,)))
    tc_v, tc_i = _tc_l1_argmin(keys, query)
    m = jnp.min(sc_v)
    mi = jnp.min(jnp.where(sc_v == m, sc_i, jnp.int32(2 ** 30)))
    pred = (tc_v[0] < m) | ((tc_v[0] == m) & (tc_i[0] < mi))
    return jnp.where(pred, tc_i[0], mi)
